# Initial kernel scaffold; baseline (speedup 1.0000x reference)
import jax
import jax.numpy as jnp
from jax.experimental import pallas as pl


def kernel(x, edge_index, edge_label_index, W1, b1, W2, b2):
    raise NotImplementedError("write your pallas kernel here")


# speedup vs baseline: 12.6694x; 12.6694x over previous
import functools

import jax
import jax.numpy as jnp
from jax import lax
from jax.experimental import pallas as pl
from jax.experimental.pallas import tpu as pltpu
from jax.experimental.pallas import tpu_sc as plsc

N_C, N_S = 2, 16
N_W = N_C * N_S
CHUNK = 80
RCHUNK = 128


def _mesh():
    return plsc.VectorSubcoreMesh(
        core_axis_name="c", subcore_axis_name="s",
        num_cores=N_C, num_subcores=N_S)


def _make_deg(N, E):
    e_per_sub = E // N_W
    n_ec = e_per_sub // CHUNK
    seg = (-(-N // N_S) + 127) // 128 * 128
    n_pad = seg * N_S

    @functools.partial(
        pl.kernel,
        out_type=jax.ShapeDtypeStruct((N_C * n_pad,), jnp.float32),
        mesh=_mesh(),
        scratch_types=[
            pltpu.VMEM((CHUNK,), jnp.int32),
            pltpu.VMEM((CHUNK,), jnp.float32),
            pltpu.VMEM((seg,), jnp.float32),
            pltpu.VMEM_SHARED((n_pad,), jnp.float32),
            pltpu.SemaphoreType.DMA,
        ],
    )
    def deg(dst_hbm, ones_hbm, zseg_hbm, out_hbm, idx_v, ones_v, seg_v, acc_sh, sem):
        c = lax.axis_index("c")
        s = lax.axis_index("s")
        pltpu.sync_copy(ones_hbm, ones_v)
        pltpu.sync_copy(zseg_hbm, seg_v)
        pltpu.sync_copy(seg_v, acc_sh.at[pl.ds(s * seg, seg)])
        plsc.subcore_barrier()
        base = (c * N_S + s) * e_per_sub

        def edge_step(i, carry):
            pltpu.sync_copy(dst_hbm.at[pl.ds(base + i * CHUNK, CHUNK)], idx_v)
            pltpu.sync_copy(ones_v, acc_sh.at[idx_v], add=True)
            return carry

        lax.fori_loop(0, n_ec, edge_step, 0)
        plsc.subcore_barrier()
        pltpu.sync_copy(acc_sh.at[pl.ds(s * seg, seg)], seg_v)
        pltpu.sync_copy(seg_v, out_hbm.at[pl.ds(c * n_pad + s * seg, seg)])

    return deg, n_pad


def _make_agg(N, E, C):
    rows_per_sub = (-(-N // N_S) + 127) // 128 * 128
    n_padr = rows_per_sub * N_S
    n_rw = rows_per_sub // RCHUNK
    e_per_sub = E // N_W
    n_ec = e_per_sub // CHUNK

    @functools.partial(
        pl.kernel,
        out_type=jax.ShapeDtypeStruct((N_C, n_padr, C), jnp.float32),
        mesh=_mesh(),
        scratch_types=[
            pltpu.VMEM((CHUNK,), jnp.int32),
            pltpu.VMEM((CHUNK,), jnp.int32),
            pltpu.VMEM((CHUNK, C), jnp.float32),
            pltpu.VMEM((RCHUNK, C), jnp.float32),
            pltpu.VMEM_SHARED((n_padr, C), jnp.float32),
            pltpu.SemaphoreType.DMA,
        ],
    )
    def agg(g_hbm, src_hbm, dst_hbm, zrows_hbm, out_hbm,
            idxs_v, idxd_v, rows_v, bounce_v, acc_sh, sem):
        c = lax.axis_index("c")
        s = lax.axis_index("s")
        row0 = s * rows_per_sub
        pltpu.sync_copy(zrows_hbm, bounce_v)
        for r in range(n_rw):
            pltpu.sync_copy(bounce_v, acc_sh.at[pl.ds(row0 + r * RCHUNK, RCHUNK)])
        plsc.subcore_barrier()
        base = (c * N_S + s) * e_per_sub

        def edge_step(i, carry):
            off = base + i * CHUNK
            pltpu.sync_copy(src_hbm.at[pl.ds(off, CHUNK)], idxs_v)
            pltpu.sync_copy(dst_hbm.at[pl.ds(off, CHUNK)], idxd_v)
            pltpu.async_copy(g_hbm.at[idxs_v], rows_v, sem).wait()
            pltpu.sync_copy(rows_v, acc_sh.at[idxd_v], add=True)
            return carry

        lax.fori_loop(0, n_ec, edge_step, 0)
        plsc.subcore_barrier()
        for r in range(n_rw):
            rr = row0 + r * RCHUNK
            pltpu.sync_copy(acc_sh.at[pl.ds(rr, RCHUNK)], bounce_v)
            pltpu.sync_copy(bounce_v, out_hbm.at[c, pl.ds(rr, RCHUNK)])

    return agg


def _make_decode(N, P, C):
    n_pc = P // CHUNK
    per = -(-n_pc // N_W)

    @functools.partial(
        pl.kernel,
        out_type=jax.ShapeDtypeStruct((P, 16), jnp.float32),
        mesh=_mesh(),
        scratch_types=[
            pltpu.VMEM((CHUNK,), jnp.int32),
            pltpu.VMEM((CHUNK,), jnp.int32),
            pltpu.VMEM((CHUNK, C), jnp.float32),
            pltpu.VMEM((CHUNK, C), jnp.float32),
            pltpu.VMEM((CHUNK, 16), jnp.float32),
            pltpu.SemaphoreType.DMA,
        ],
    )
    def dec(z_hbm, ea_hbm, eb_hbm, out_hbm, ia_v, ib_v, za_v, zb_v, sc_v, sem):
        c = lax.axis_index("c")
        s = lax.axis_index("s")
        wid = c * N_S + s

        def chunk_step(t, carry):
            ci = wid + t * N_W

            @pl.when(ci < n_pc)
            def _():
                off = ci * CHUNK
                pltpu.sync_copy(ea_hbm.at[pl.ds(off, CHUNK)], ia_v)
                pltpu.sync_copy(eb_hbm.at[pl.ds(off, CHUNK)], ib_v)
                pltpu.async_copy(z_hbm.at[ia_v], za_v, sem).wait()
                pltpu.async_copy(z_hbm.at[ib_v], zb_v, sem).wait()

                def pair_step(p, carry2):
                    v = za_v[p, pl.ds(0, 16)] * zb_v[p, pl.ds(0, 16)]
                    for k in range(1, C // 16):
                        v = v + (za_v[p, pl.ds(k * 16, 16)]
                                 * zb_v[p, pl.ds(k * 16, 16)])
                    sc_v[p, pl.ds(0, 16)] = v
                    return carry2

                lax.fori_loop(0, CHUNK, pair_step, 0)
                pltpu.sync_copy(sc_v, out_hbm.at[pl.ds(off, CHUNK)])

            return carry

        lax.fori_loop(0, per, chunk_step, 0)

    return dec


def _tc4_body(ps_ref, o_ref):
    o_ref[...] = jnp.sum(ps_ref[...], axis=-1, keepdims=True)


def _tc1_body(cnta_ref, cntb_ref, x_ref, w_ref, o_ref):
    dinv = lax.rsqrt(cnta_ref[...] + cntb_ref[...] + 1.0)
    o_ref[...] = jnp.dot(x_ref[...], w_ref[...],
                         preferred_element_type=jnp.float32) * dinv


def _tc2_body(part_ref, g_ref, cnta_ref, cntb_ref, b1_ref, w2_ref, o_ref):
    dinv = lax.rsqrt(cnta_ref[...] + cntb_ref[...] + 1.0)
    t = (part_ref[0] + part_ref[1] + g_ref[...]) * dinv + b1_ref[...]
    t = jnp.maximum(t, 0.0)
    o_ref[...] = jnp.dot(t, w2_ref[...],
                         preferred_element_type=jnp.float32) * dinv


def _tc3_body(part_ref, g_ref, cnta_ref, cntb_ref, b2_ref, o_ref):
    dinv = lax.rsqrt(cnta_ref[...] + cntb_ref[...] + 1.0)
    o_ref[...] = (part_ref[0] + part_ref[1] + g_ref[...]) * dinv + b2_ref[...]


def kernel(x, edge_index, edge_label_index, W1, b1, W2, b2):
    N, C = x.shape
    E = edge_index.shape[1]
    P = edge_label_index.shape[1]
    B = 2000
    grid = (N // B,)

    src, dst = edge_index[0], edge_index[1]
    ea, eb = edge_label_index[0], edge_label_index[1]

    deg_call, n_pad = _make_deg(N, E)
    agg_call = _make_agg(N, E, C)
    dec_call = _make_decode(N, P, C)

    ones80 = jnp.ones((CHUNK,), jnp.float32)
    zseg = jnp.zeros((n_pad // N_S,), jnp.float32)
    zrows = jnp.zeros((RCHUNK, C), jnp.float32)

    cnt = deg_call(dst, ones80, zseg).reshape(N_C, n_pad)
    cnta = cnt[0, :N].reshape(N, 1)
    cntb = cnt[1, :N].reshape(N, 1)

    col = pl.BlockSpec((B, 1), lambda i: (i, 0))
    mat = pl.BlockSpec((B, C), lambda i: (i, 0))
    wts = pl.BlockSpec((C, C), lambda i: (0, 0))
    bias = pl.BlockSpec((1, C), lambda i: (0, 0))
    parts = pl.BlockSpec((N_C, B, C), lambda i: (0, i, 0))
    out_sds = jax.ShapeDtypeStruct((N, C), jnp.float32)

    g1 = pl.pallas_call(
        _tc1_body, grid=grid,
        in_specs=[col, col, mat, wts],
        out_specs=mat, out_shape=out_sds,
    )(cnta, cntb, x, W1)

    part1 = agg_call(g1, src, dst, zrows)

    g2 = pl.pallas_call(
        _tc2_body, grid=grid,
        in_specs=[parts, mat, col, col, bias, wts],
        out_specs=mat, out_shape=out_sds,
    )(part1, g1, cnta, cntb, b1.reshape(1, C), W2)

    part2 = agg_call(g2, src, dst, zrows)

    z = pl.pallas_call(
        _tc3_body, grid=grid,
        in_specs=[parts, mat, col, col, bias],
        out_specs=mat, out_shape=out_sds,
    )(part2, g2, cnta, cntb, b2.reshape(1, C))

    partial_dots = dec_call(z, ea, eb)

    scores = pl.pallas_call(
        _tc4_body, grid=(P // B,),
        in_specs=[pl.BlockSpec((B, 16), lambda i: (i, 0))],
        out_specs=pl.BlockSpec((B, 1), lambda i: (i, 0)),
        out_shape=jax.ShapeDtypeStruct((P, 1), jnp.float32),
    )(partial_dots)
    return scores.reshape(P)


# speedup vs baseline: 16.8342x; 1.3287x over previous
import functools

import jax
import jax.numpy as jnp
from jax import lax
from jax.experimental import pallas as pl
from jax.experimental.pallas import tpu as pltpu
from jax.experimental.pallas import tpu_sc as plsc

N_C, N_S = 2, 16
N_W = N_C * N_S
CHUNK = 80
ECH = 32
N_BUF = 5


def _mesh():
    return plsc.VectorSubcoreMesh(
        core_axis_name="c", subcore_axis_name="s",
        num_cores=N_C, num_subcores=N_S)


def _make_deg(N, E):
    e_per_sub = E // N_W
    n_ec = e_per_sub // ECH
    seg = (-(-N // N_S) + 127) // 128 * 128
    n_pad = seg * N_S

    @functools.partial(
        pl.kernel,
        out_type=jax.ShapeDtypeStruct((N_C * n_pad,), jnp.float32),
        mesh=_mesh(),
        scratch_types=[
            pltpu.VMEM((n_ec, ECH), jnp.int32),
            pltpu.VMEM((ECH,), jnp.float32),
            pltpu.VMEM((seg,), jnp.float32),
            pltpu.VMEM_SHARED((n_pad,), jnp.float32),
            pltpu.SemaphoreType.DMA,
        ],
    )
    def deg(dst3_hbm, ones_hbm, zseg_hbm, out_hbm, idx_v, ones_v, seg_v, acc_sh, sem):
        c = lax.axis_index("c")
        s = lax.axis_index("s")
        w = c * N_S + s
        pltpu.sync_copy(dst3_hbm.at[w], idx_v)
        pltpu.sync_copy(ones_hbm, ones_v)
        pltpu.sync_copy(zseg_hbm, seg_v)
        pltpu.sync_copy(seg_v, acc_sh.at[pl.ds(s * seg, seg)])
        plsc.subcore_barrier()

        def fire(i, carry):
            pltpu.async_copy(ones_v, acc_sh.at[idx_v.at[i]], sem, add=True)
            return carry

        lax.fori_loop(0, n_ec, fire, 0)

        def drain(i, carry):
            pltpu.make_async_copy(ones_v, acc_sh.at[pl.ds(0, ECH)], sem).wait()
            return carry

        lax.fori_loop(0, n_ec, drain, 0)
        plsc.subcore_barrier()
        pltpu.sync_copy(acc_sh.at[pl.ds(s * seg, seg)], seg_v)
        pltpu.sync_copy(seg_v, out_hbm.at[pl.ds(c * n_pad + s * seg, seg)])

    return deg, n_pad


def _make_agg(N, E, C):
    rows_per_sub = (-(-N // N_S) + 127) // 128 * 128
    n_padr = rows_per_sub * N_S
    e_per_sub = E // N_W
    n_ec = e_per_sub // ECH

    @functools.partial(
        pl.kernel,
        out_type=jax.ShapeDtypeStruct((N_C, n_padr, C), jnp.float32),
        mesh=_mesh(),
        scratch_types=[
            pltpu.VMEM((e_per_sub,), jnp.int32),
            pltpu.VMEM((e_per_sub,), jnp.int32),
            [pltpu.VMEM((ECH, C), jnp.float32) for _ in range(N_BUF)],
            pltpu.VMEM_SHARED((n_padr, C), jnp.float32),
            [pltpu.SemaphoreType.DMA for _ in range(N_BUF)],
            pltpu.SemaphoreType.DMA,
            pltpu.SemaphoreType.DMA,
        ],
    )
    def agg(g_hbm, src_hbm, dst_hbm, zrows_hbm, out_hbm,
            sidx_v, didx_v, rows_v, acc_sh, gsem, ssem_a, ssem_b):
        c = lax.axis_index("c")
        s = lax.axis_index("s")
        w = c * N_S + s
        row0 = s * rows_per_sub
        pltpu.sync_copy(src_hbm.at[pl.ds(w * e_per_sub, e_per_sub)], sidx_v)
        pltpu.sync_copy(dst_hbm.at[pl.ds(w * e_per_sub, e_per_sub)], didx_v)
        pltpu.sync_copy(zrows_hbm, acc_sh.at[pl.ds(row0, rows_per_sub)])
        plsc.subcore_barrier()

        for b in range(N_BUF):
            pltpu.async_copy(
                g_hbm.at[sidx_v.at[pl.ds(b * ECH, ECH)]], rows_v[b], gsem[b])

        def outer(o, carry):
            for b in range(N_BUF):
                ci = o * N_BUF + b
                pltpu.make_async_copy(
                    g_hbm.at[pl.ds(0, ECH)], rows_v[b], gsem[b]).wait()
                ia = didx_v[pl.ds(ci * ECH, 16)]
                ib = didx_v[pl.ds(ci * ECH + 16, 16)]
                da = pltpu.async_copy(
                    rows_v[b].at[pl.ds(0, 16)], acc_sh.at[ia], ssem_a, add=True)
                db = pltpu.async_copy(
                    rows_v[b].at[pl.ds(16, 16)], acc_sh.at[ib], ssem_b, add=True)
                da.wait()
                db.wait()
                nxt = ci + N_BUF

                @pl.when(nxt < n_ec)
                def _():
                    pltpu.async_copy(
                        g_hbm.at[sidx_v.at[pl.ds(nxt * ECH, ECH)]],
                        rows_v[b], gsem[b])

            return carry

        lax.fori_loop(0, n_ec // N_BUF, outer, 0)
        plsc.subcore_barrier()
        pltpu.sync_copy(acc_sh.at[pl.ds(row0, rows_per_sub)],
                        out_hbm.at[c, pl.ds(row0, rows_per_sub)])

    return agg


def _make_decode(N, P, C):
    n_pc = P // CHUNK
    per = -(-n_pc // N_W)

    @functools.partial(
        pl.kernel,
        out_type=jax.ShapeDtypeStruct((P, 16), jnp.float32),
        mesh=_mesh(),
        scratch_types=[
            pltpu.VMEM((CHUNK,), jnp.int32),
            pltpu.VMEM((CHUNK,), jnp.int32),
            pltpu.VMEM((CHUNK, C), jnp.float32),
            pltpu.VMEM((CHUNK, C), jnp.float32),
            pltpu.VMEM((CHUNK, 16), jnp.float32),
            pltpu.SemaphoreType.DMA,
            pltpu.SemaphoreType.DMA,
        ],
    )
    def dec(z_hbm, ea_hbm, eb_hbm, out_hbm, ia_v, ib_v, za_v, zb_v, sc_v, sem, sem2):
        c = lax.axis_index("c")
        s = lax.axis_index("s")
        wid = c * N_S + s

        def chunk_step(t, carry):
            ci = wid + t * N_W

            @pl.when(ci < n_pc)
            def _():
                off = ci * CHUNK
                pltpu.sync_copy(ea_hbm.at[pl.ds(off, CHUNK)], ia_v)
                pltpu.sync_copy(eb_hbm.at[pl.ds(off, CHUNK)], ib_v)
                da = pltpu.async_copy(z_hbm.at[ia_v], za_v, sem)
                db = pltpu.async_copy(z_hbm.at[ib_v], zb_v, sem2)
                da.wait()
                db.wait()

                def pair_step(p, carry2):
                    v = za_v[p, pl.ds(0, 16)] * zb_v[p, pl.ds(0, 16)]
                    for k in range(1, C // 16):
                        v = v + (za_v[p, pl.ds(k * 16, 16)]
                                 * zb_v[p, pl.ds(k * 16, 16)])
                    sc_v[p, pl.ds(0, 16)] = v
                    return carry2

                lax.fori_loop(0, CHUNK, pair_step, 0)
                pltpu.sync_copy(sc_v, out_hbm.at[pl.ds(off, CHUNK)])

            return carry

        lax.fori_loop(0, per, chunk_step, 0)

    return dec


def _tc4_body(ps_ref, o_ref):
    o_ref[...] = jnp.sum(ps_ref[...], axis=-1, keepdims=True)


def _tc1_body(cnta_ref, cntb_ref, x_ref, w_ref, o_ref):
    dinv = lax.rsqrt(cnta_ref[...] + cntb_ref[...] + 1.0)
    o_ref[...] = jnp.dot(x_ref[...], w_ref[...],
                         preferred_element_type=jnp.float32) * dinv


def _tc2_body(part_ref, g_ref, cnta_ref, cntb_ref, b1_ref, w2_ref, o_ref):
    dinv = lax.rsqrt(cnta_ref[...] + cntb_ref[...] + 1.0)
    t = (part_ref[0] + part_ref[1] + g_ref[...]) * dinv + b1_ref[...]
    t = jnp.maximum(t, 0.0)
    o_ref[...] = jnp.dot(t, w2_ref[...],
                         preferred_element_type=jnp.float32) * dinv


def _tc3_body(part_ref, g_ref, cnta_ref, cntb_ref, b2_ref, o_ref):
    dinv = lax.rsqrt(cnta_ref[...] + cntb_ref[...] + 1.0)
    o_ref[...] = (part_ref[0] + part_ref[1] + g_ref[...]) * dinv + b2_ref[...]


def kernel(x, edge_index, edge_label_index, W1, b1, W2, b2):
    N, C = x.shape
    E = edge_index.shape[1]
    P = edge_label_index.shape[1]
    B = 2000
    grid = (N // B,)

    e_per_sub = -(-E // (N_W * ECH * N_BUF)) * ECH * N_BUF
    e_pad = N_W * e_per_sub - E
    pad_row = (-(-N // N_S) + 127) // 128 * 128 * N_S - 1
    src_f = jnp.concatenate(
        [edge_index[0], jnp.zeros((e_pad,), jnp.int32)])
    dst_f = jnp.concatenate(
        [edge_index[1], jnp.full((e_pad,), pad_row, jnp.int32)])
    dst3 = dst_f.reshape(N_W, e_per_sub // ECH, ECH)
    ea, eb = edge_label_index[0], edge_label_index[1]

    E_p = N_W * e_per_sub
    deg_call, n_pad = _make_deg(N, E_p)
    agg_call = _make_agg(N, E_p, C)
    dec_call = _make_decode(N, P, C)

    ones_e = jnp.ones((ECH,), jnp.float32)
    zseg = jnp.zeros((n_pad // N_S,), jnp.float32)
    zrows = jnp.zeros((n_pad // N_S, C), jnp.float32)

    cnt = deg_call(dst3, ones_e, zseg).reshape(N_C, n_pad)
    cnta = cnt[0, :N].reshape(N, 1)
    cntb = cnt[1, :N].reshape(N, 1)

    col = pl.BlockSpec((B, 1), lambda i: (i, 0))
    mat = pl.BlockSpec((B, C), lambda i: (i, 0))
    wts = pl.BlockSpec((C, C), lambda i: (0, 0))
    bias = pl.BlockSpec((1, C), lambda i: (0, 0))
    parts = pl.BlockSpec((N_C, B, C), lambda i: (0, i, 0))
    out_sds = jax.ShapeDtypeStruct((N, C), jnp.float32)

    g1 = pl.pallas_call(
        _tc1_body, grid=grid,
        in_specs=[col, col, mat, wts],
        out_specs=mat, out_shape=out_sds,
    )(cnta, cntb, x, W1)

    part1 = agg_call(g1, src_f, dst_f, zrows)

    g2 = pl.pallas_call(
        _tc2_body, grid=grid,
        in_specs=[parts, mat, col, col, bias, wts],
        out_specs=mat, out_shape=out_sds,
    )(part1, g1, cnta, cntb, b1.reshape(1, C), W2)

    part2 = agg_call(g2, src_f, dst_f, zrows)

    z = pl.pallas_call(
        _tc3_body, grid=grid,
        in_specs=[parts, mat, col, col, bias],
        out_specs=mat, out_shape=out_sds,
    )(part2, g2, cnta, cntb, b2.reshape(1, C))

    partial_dots = dec_call(z, ea, eb)

    scores = pl.pallas_call(
        _tc4_body, grid=(P // B,),
        in_specs=[pl.BlockSpec((B, 16), lambda i: (i, 0))],
        out_specs=pl.BlockSpec((B, 1), lambda i: (i, 0)),
        out_shape=jax.ShapeDtypeStruct((P, 1), jnp.float32),
    )(partial_dots)
    return scores.reshape(P)


# speedup vs baseline: 18.7933x; 1.1164x over previous
import functools

import jax
import jax.numpy as jnp
from jax import lax
from jax.experimental import pallas as pl
from jax.experimental.pallas import tpu as pltpu
from jax.experimental.pallas import tpu_sc as plsc

N_C, N_S = 2, 16
N_W = N_C * N_S
CHUNK = 80
ECH = 32
N_BUF = 5


def _mesh():
    return plsc.VectorSubcoreMesh(
        core_axis_name="c", subcore_axis_name="s",
        num_cores=N_C, num_subcores=N_S)


def _make_deg(N, E):
    e_per_sub = E // N_W
    n_ec = e_per_sub // ECH
    seg = (-(-N // N_S) + 127) // 128 * 128
    n_pad = seg * N_S

    @functools.partial(
        pl.kernel,
        out_type=jax.ShapeDtypeStruct((N_C * n_pad,), jnp.float32),
        mesh=_mesh(),
        scratch_types=[
            pltpu.VMEM((n_ec, ECH), jnp.int32),
            pltpu.VMEM((ECH,), jnp.float32),
            pltpu.VMEM((seg,), jnp.float32),
            pltpu.VMEM_SHARED((n_pad,), jnp.float32),
            pltpu.SemaphoreType.DMA,
        ],
    )
    def deg(dst3_hbm, ones_hbm, zseg_hbm, out_hbm, idx_v, ones_v, seg_v, acc_sh, sem):
        c = lax.axis_index("c")
        s = lax.axis_index("s")
        w = c * N_S + s
        pltpu.sync_copy(dst3_hbm.at[w], idx_v)
        pltpu.sync_copy(ones_hbm, ones_v)
        pltpu.sync_copy(zseg_hbm, seg_v)
        pltpu.sync_copy(seg_v, acc_sh.at[pl.ds(s * seg, seg)])
        plsc.subcore_barrier()

        def fire(i, carry):
            pltpu.async_copy(ones_v, acc_sh.at[idx_v.at[i]], sem, add=True)
            return carry

        lax.fori_loop(0, n_ec, fire, 0)

        def drain(i, carry):
            pltpu.make_async_copy(ones_v, acc_sh.at[pl.ds(0, ECH)], sem).wait()
            return carry

        lax.fori_loop(0, n_ec, drain, 0)
        plsc.subcore_barrier()
        pltpu.sync_copy(acc_sh.at[pl.ds(s * seg, seg)], seg_v)
        pltpu.sync_copy(seg_v, out_hbm.at[pl.ds(c * n_pad + s * seg, seg)])

    return deg, n_pad


def _make_agg(N, e0ps, e1ps, C):
    rows_per_sub = (-(-N // N_S) + 127) // 128 * 128
    n_padr = rows_per_sub * N_S

    @functools.partial(
        pl.kernel,
        out_type=jax.ShapeDtypeStruct((N_C, n_padr, C), jnp.float32),
        mesh=_mesh(),
        scratch_types=[
            pltpu.VMEM((max(e0ps, e1ps),), jnp.int32),
            pltpu.VMEM((max(e0ps, e1ps),), jnp.int32),
            [pltpu.VMEM((ECH, C), jnp.float32) for _ in range(N_BUF)],
            pltpu.VMEM_SHARED((n_padr, C), jnp.float32),
            [pltpu.SemaphoreType.DMA for _ in range(N_BUF)],
            pltpu.SemaphoreType.DMA,
            pltpu.SemaphoreType.DMA,
        ],
    )
    def agg(g_hbm, src_hbm, dst_hbm, zrows_hbm, out_hbm,
            sidx_v, didx_v, rows_v, acc_sh, gsem, ssem_a, ssem_b):
        c = lax.axis_index("c")
        s = lax.axis_index("s")
        row0 = s * rows_per_sub
        n_ec = jnp.where(c == 0, e0ps // ECH, e1ps // ECH)

        @pl.when(c == 0)
        def _():
            base = s * e0ps
            pltpu.sync_copy(src_hbm.at[pl.ds(base, e0ps)],
                            sidx_v.at[pl.ds(0, e0ps)])
            pltpu.sync_copy(dst_hbm.at[pl.ds(base, e0ps)],
                            didx_v.at[pl.ds(0, e0ps)])

        @pl.when(c == 1)
        def _():
            base = N_S * e0ps + s * e1ps
            pltpu.sync_copy(src_hbm.at[pl.ds(base, e1ps)],
                            sidx_v.at[pl.ds(0, e1ps)])
            pltpu.sync_copy(dst_hbm.at[pl.ds(base, e1ps)],
                            didx_v.at[pl.ds(0, e1ps)])

        pltpu.sync_copy(zrows_hbm, acc_sh.at[pl.ds(row0, rows_per_sub)])
        plsc.subcore_barrier()

        for b in range(N_BUF):
            pltpu.async_copy(
                g_hbm.at[sidx_v.at[pl.ds(b * ECH, ECH)]], rows_v[b], gsem[b])

        def outer(o, carry):
            for b in range(N_BUF):
                ci = o * N_BUF + b
                pltpu.make_async_copy(
                    g_hbm.at[pl.ds(0, ECH)], rows_v[b], gsem[b]).wait()
                ia = didx_v[pl.ds(ci * ECH, 16)]
                ib = didx_v[pl.ds(ci * ECH + 16, 16)]
                da = pltpu.async_copy(
                    rows_v[b].at[pl.ds(0, 16)], acc_sh.at[ia], ssem_a, add=True)
                db = pltpu.async_copy(
                    rows_v[b].at[pl.ds(16, 16)], acc_sh.at[ib], ssem_b, add=True)
                da.wait()
                db.wait()
                nxt = ci + N_BUF

                @pl.when(nxt < n_ec)
                def _():
                    pltpu.async_copy(
                        g_hbm.at[sidx_v.at[pl.ds(nxt * ECH, ECH)]],
                        rows_v[b], gsem[b])

            return carry

        lax.fori_loop(0, n_ec // N_BUF, outer, 0)
        plsc.subcore_barrier()
        pltpu.sync_copy(acc_sh.at[pl.ds(row0, rows_per_sub)],
                        out_hbm.at[c, pl.ds(row0, rows_per_sub)])

    return agg


def _make_decode(N, P, C):
    n_pc = P // CHUNK
    per = -(-n_pc // N_W)

    @functools.partial(
        pl.kernel,
        out_type=jax.ShapeDtypeStruct((P, 16), jnp.float32),
        mesh=_mesh(),
        scratch_types=[
            pltpu.VMEM((CHUNK,), jnp.int32),
            pltpu.VMEM((CHUNK,), jnp.int32),
            pltpu.VMEM((CHUNK, C), jnp.float32),
            pltpu.VMEM((CHUNK, C), jnp.float32),
            pltpu.VMEM((CHUNK, 16), jnp.float32),
            pltpu.SemaphoreType.DMA,
            pltpu.SemaphoreType.DMA,
        ],
    )
    def dec(z_hbm, ea_hbm, eb_hbm, out_hbm, ia_v, ib_v, za_v, zb_v, sc_v, sem, sem2):
        c = lax.axis_index("c")
        s = lax.axis_index("s")
        wid = c * N_S + s

        def chunk_step(t, carry):
            ci = wid + t * N_W

            @pl.when(ci < n_pc)
            def _():
                off = ci * CHUNK
                pltpu.sync_copy(ea_hbm.at[pl.ds(off, CHUNK)], ia_v)
                pltpu.sync_copy(eb_hbm.at[pl.ds(off, CHUNK)], ib_v)
                da = pltpu.async_copy(z_hbm.at[ia_v], za_v, sem)
                db = pltpu.async_copy(z_hbm.at[ib_v], zb_v, sem2)
                da.wait()
                db.wait()

                def pair_step(p, carry2):
                    v = za_v[p, pl.ds(0, 16)] * zb_v[p, pl.ds(0, 16)]
                    for k in range(1, C // 16):
                        v = v + (za_v[p, pl.ds(k * 16, 16)]
                                 * zb_v[p, pl.ds(k * 16, 16)])
                    sc_v[p, pl.ds(0, 16)] = v
                    return carry2

                lax.fori_loop(0, CHUNK, pair_step, 0)
                pltpu.sync_copy(sc_v, out_hbm.at[pl.ds(off, CHUNK)])

            return carry

        lax.fori_loop(0, per, chunk_step, 0)

    return dec


def _tc4_body(ps_ref, o_ref):
    o_ref[...] = jnp.sum(ps_ref[...], axis=-1, keepdims=True)


def _tc1_body(cnta_ref, cntb_ref, x_ref, w_ref, o_ref):
    dinv = lax.rsqrt(cnta_ref[...] + cntb_ref[...] + 1.0)
    o_ref[...] = jnp.dot(x_ref[...], w_ref[...],
                         preferred_element_type=jnp.float32) * dinv


def _tc2_body(part_ref, g_ref, cnta_ref, cntb_ref, b1_ref, w2_ref, o_ref):
    dinv = lax.rsqrt(cnta_ref[...] + cntb_ref[...] + 1.0)
    t = (part_ref[0] + part_ref[1] + g_ref[...]) * dinv + b1_ref[...]
    t = jnp.maximum(t, 0.0)
    o_ref[...] = jnp.dot(t, w2_ref[...],
                         preferred_element_type=jnp.float32) * dinv


def _tc3_body(part_ref, g_ref, cnta_ref, cntb_ref, b2_ref, o_ref):
    dinv = lax.rsqrt(cnta_ref[...] + cntb_ref[...] + 1.0)
    o_ref[...] = (part_ref[0] + part_ref[1] + g_ref[...]) * dinv + b2_ref[...]


def kernel(x, edge_index, edge_label_index, W1, b1, W2, b2):
    N, C = x.shape
    E = edge_index.shape[1]
    P = edge_label_index.shape[1]
    B = 2000
    grid = (N // B,)

    quant = ECH * N_BUF
    t_ps = -(-E // (N_W * quant)) * quant
    e_pad = N_W * t_ps - E
    pad_row = (-(-N // N_S) + 127) // 128 * 128 * N_S - 1
    src_f = jnp.concatenate(
        [edge_index[0], jnp.zeros((e_pad,), jnp.int32)])
    dst_f = jnp.concatenate(
        [edge_index[1], jnp.full((e_pad,), pad_row, jnp.int32)])
    E_p = N_W * t_ps
    dst3 = dst_f.reshape(N_W, t_ps // ECH, ECH)
    ea, eb = edge_label_index[0], edge_label_index[1]

    e0ps = int(round(2 * t_ps * 0.70 / quant)) * quant
    e1ps = 2 * t_ps - e0ps

    deg_call, n_pad = _make_deg(N, E_p)
    agg_call = _make_agg(N, e0ps, e1ps, C)
    dec_call = _make_decode(N, P, C)

    ones_e = jnp.ones((ECH,), jnp.float32)
    zseg = jnp.zeros((n_pad // N_S,), jnp.float32)
    zrows = jnp.zeros((n_pad // N_S, C), jnp.float32)

    cnt = deg_call(dst3, ones_e, zseg).reshape(N_C, n_pad)
    cnta = cnt[0, :N].reshape(N, 1)
    cntb = cnt[1, :N].reshape(N, 1)

    col = pl.BlockSpec((B, 1), lambda i: (i, 0))
    mat = pl.BlockSpec((B, C), lambda i: (i, 0))
    wts = pl.BlockSpec((C, C), lambda i: (0, 0))
    bias = pl.BlockSpec((1, C), lambda i: (0, 0))
    parts = pl.BlockSpec((N_C, B, C), lambda i: (0, i, 0))
    out_sds = jax.ShapeDtypeStruct((N, C), jnp.float32)

    g1 = pl.pallas_call(
        _tc1_body, grid=grid,
        in_specs=[col, col, mat, wts],
        out_specs=mat, out_shape=out_sds,
    )(cnta, cntb, x, W1)

    part1 = agg_call(g1, src_f, dst_f, zrows)

    g2 = pl.pallas_call(
        _tc2_body, grid=grid,
        in_specs=[parts, mat, col, col, bias, wts],
        out_specs=mat, out_shape=out_sds,
    )(part1, g1, cnta, cntb, b1.reshape(1, C), W2)

    part2 = agg_call(g2, src_f, dst_f, zrows)

    z = pl.pallas_call(
        _tc3_body, grid=grid,
        in_specs=[parts, mat, col, col, bias],
        out_specs=mat, out_shape=out_sds,
    )(part2, g2, cnta, cntb, b2.reshape(1, C))

    partial_dots = dec_call(z, ea, eb)

    scores = pl.pallas_call(
        _tc4_body, grid=(P // B,),
        in_specs=[pl.BlockSpec((B, 16), lambda i: (i, 0))],
        out_specs=pl.BlockSpec((B, 1), lambda i: (i, 0)),
        out_shape=jax.ShapeDtypeStruct((P, 1), jnp.float32),
    )(partial_dots)
    return scores.reshape(P)


# speedup vs baseline: 20.1534x; 1.0724x over previous
import functools

import jax
import jax.numpy as jnp
from jax import lax
from jax.experimental import pallas as pl
from jax.experimental.pallas import tpu as pltpu
from jax.experimental.pallas import tpu_sc as plsc

N_C, N_S = 2, 16
N_W = N_C * N_S
CHUNK = 80
ECH = 32
N_BUF = 5


def _mesh():
    return plsc.VectorSubcoreMesh(
        core_axis_name="c", subcore_axis_name="s",
        num_cores=N_C, num_subcores=N_S)


def _make_deg(N, E):
    e_per_sub = E // N_W
    n_ec = e_per_sub // ECH
    seg = (-(-N // N_S) + 127) // 128 * 128
    n_pad = seg * N_S

    @functools.partial(
        pl.kernel,
        out_type=jax.ShapeDtypeStruct((N_C * n_pad,), jnp.float32),
        mesh=_mesh(),
        scratch_types=[
            pltpu.VMEM((n_ec, ECH), jnp.int32),
            pltpu.VMEM((ECH,), jnp.float32),
            pltpu.VMEM((seg,), jnp.float32),
            pltpu.VMEM_SHARED((n_pad,), jnp.float32),
            pltpu.SemaphoreType.DMA,
        ],
    )
    def deg(dst3_hbm, ones_hbm, zseg_hbm, out_hbm, idx_v, ones_v, seg_v, acc_sh, sem):
        c = lax.axis_index("c")
        s = lax.axis_index("s")
        w = c * N_S + s
        pltpu.sync_copy(dst3_hbm.at[w], idx_v)
        pltpu.sync_copy(ones_hbm, ones_v)
        pltpu.sync_copy(zseg_hbm, seg_v)
        pltpu.sync_copy(seg_v, acc_sh.at[pl.ds(s * seg, seg)])
        plsc.subcore_barrier()

        def fire(i, carry):
            pltpu.async_copy(ones_v, acc_sh.at[idx_v.at[i]], sem, add=True)
            return carry

        lax.fori_loop(0, n_ec, fire, 0)

        def drain(i, carry):
            pltpu.make_async_copy(ones_v, acc_sh.at[pl.ds(0, ECH)], sem).wait()
            return carry

        lax.fori_loop(0, n_ec, drain, 0)
        plsc.subcore_barrier()
        pltpu.sync_copy(acc_sh.at[pl.ds(s * seg, seg)], seg_v)
        pltpu.sync_copy(seg_v, out_hbm.at[pl.ds(c * n_pad + s * seg, seg)])

    return deg, n_pad


def _make_agg(N, e0ps, e1ps, C):
    rows_per_sub = (-(-N // N_S) + 127) // 128 * 128
    n_padr = rows_per_sub * N_S

    @functools.partial(
        pl.kernel,
        out_type=jax.ShapeDtypeStruct((N_C, n_padr, C), jnp.float32),
        mesh=_mesh(),
        scratch_types=[
            pltpu.VMEM((max(e0ps, e1ps),), jnp.int32),
            pltpu.VMEM((max(e0ps, e1ps),), jnp.int32),
            [pltpu.VMEM((ECH, C), jnp.float32) for _ in range(N_BUF)],
            pltpu.VMEM_SHARED((n_padr, C), jnp.float32),
            [pltpu.SemaphoreType.DMA for _ in range(N_BUF)],
            pltpu.SemaphoreType.DMA,
            pltpu.SemaphoreType.DMA,
        ],
    )
    def agg(g_hbm, src_hbm, dst_hbm, out_hbm,
            sidx_v, didx_v, rows_v, acc_sh, gsem, ssem_a, ssem_b):
        c = lax.axis_index("c")
        s = lax.axis_index("s")
        row0 = s * rows_per_sub
        n_ec = jnp.where(c == 0, e0ps // ECH, e1ps // ECH)

        @pl.when(c == 0)
        def _():
            base = s * e0ps
            pltpu.sync_copy(src_hbm.at[pl.ds(base, e0ps)],
                            sidx_v.at[pl.ds(0, e0ps)])
            pltpu.sync_copy(dst_hbm.at[pl.ds(base, e0ps)],
                            didx_v.at[pl.ds(0, e0ps)])
            pltpu.sync_copy(g_hbm.at[pl.ds(row0, rows_per_sub)],
                            acc_sh.at[pl.ds(row0, rows_per_sub)])

        @pl.when(c == 1)
        def _():
            base = N_S * e0ps + s * e1ps
            pltpu.sync_copy(src_hbm.at[pl.ds(base, e1ps)],
                            sidx_v.at[pl.ds(0, e1ps)])
            pltpu.sync_copy(dst_hbm.at[pl.ds(base, e1ps)],
                            didx_v.at[pl.ds(0, e1ps)])
            zv = jnp.zeros((16,), jnp.float32)

            def zrow(i, carry):
                for k in range(C // 16):
                    rows_v[0][i, pl.ds(k * 16, 16)] = zv
                return carry

            lax.fori_loop(0, ECH, zrow, 0)
            for j in range(rows_per_sub // ECH):
                pltpu.sync_copy(rows_v[0],
                                acc_sh.at[pl.ds(row0 + j * ECH, ECH)])

        plsc.subcore_barrier()

        for b in range(N_BUF):
            pltpu.async_copy(
                g_hbm.at[sidx_v.at[pl.ds(b * ECH, ECH)]], rows_v[b], gsem[b])

        def outer(o, carry):
            for b in range(N_BUF):
                ci = o * N_BUF + b
                pltpu.make_async_copy(
                    g_hbm.at[pl.ds(0, ECH)], rows_v[b], gsem[b]).wait()
                ia = didx_v[pl.ds(ci * ECH, 16)]
                ib = didx_v[pl.ds(ci * ECH + 16, 16)]
                da = pltpu.async_copy(
                    rows_v[b].at[pl.ds(0, 16)], acc_sh.at[ia], ssem_a, add=True)
                db = pltpu.async_copy(
                    rows_v[b].at[pl.ds(16, 16)], acc_sh.at[ib], ssem_b, add=True)
                da.wait()
                db.wait()
                nxt = ci + N_BUF

                @pl.when(nxt < n_ec)
                def _():
                    pltpu.async_copy(
                        g_hbm.at[sidx_v.at[pl.ds(nxt * ECH, ECH)]],
                        rows_v[b], gsem[b])

            return carry

        lax.fori_loop(0, n_ec // N_BUF, outer, 0)
        plsc.subcore_barrier()
        pltpu.sync_copy(acc_sh.at[pl.ds(row0, rows_per_sub)],
                        out_hbm.at[c, pl.ds(row0, rows_per_sub)])

    return agg


def _make_decode(N, P, C):
    n_pc = P // CHUNK
    per = -(-n_pc // N_W)

    @functools.partial(
        pl.kernel,
        out_type=jax.ShapeDtypeStruct((P, 16), jnp.float32),
        mesh=_mesh(),
        scratch_types=[
            pltpu.VMEM((CHUNK,), jnp.int32),
            pltpu.VMEM((CHUNK,), jnp.int32),
            pltpu.VMEM((CHUNK, C), jnp.float32),
            pltpu.VMEM((CHUNK, C), jnp.float32),
            pltpu.VMEM((CHUNK, 16), jnp.float32),
            pltpu.SemaphoreType.DMA,
            pltpu.SemaphoreType.DMA,
        ],
    )
    def dec(z_hbm, ea_hbm, eb_hbm, out_hbm, ia_v, ib_v, za_v, zb_v, sc_v, sem, sem2):
        c = lax.axis_index("c")
        s = lax.axis_index("s")
        wid = c * N_S + s

        def chunk_step(t, carry):
            ci = wid + t * N_W

            @pl.when(ci < n_pc)
            def _():
                off = ci * CHUNK
                pltpu.sync_copy(ea_hbm.at[pl.ds(off, CHUNK)], ia_v)
                pltpu.sync_copy(eb_hbm.at[pl.ds(off, CHUNK)], ib_v)
                da = pltpu.async_copy(z_hbm.at[ia_v], za_v, sem)
                db = pltpu.async_copy(z_hbm.at[ib_v], zb_v, sem2)
                da.wait()
                db.wait()

                def pair_step(p, carry2):
                    v = za_v[p, pl.ds(0, 16)] * zb_v[p, pl.ds(0, 16)]
                    for k in range(1, C // 16):
                        v = v + (za_v[p, pl.ds(k * 16, 16)]
                                 * zb_v[p, pl.ds(k * 16, 16)])
                    sc_v[p, pl.ds(0, 16)] = v
                    return carry2

                lax.fori_loop(0, CHUNK, pair_step, 0)
                pltpu.sync_copy(sc_v, out_hbm.at[pl.ds(off, CHUNK)])

            return carry

        lax.fori_loop(0, per, chunk_step, 0)

    return dec


def _tc4_body(ps_ref, o_ref):
    o_ref[...] = jnp.sum(ps_ref[...], axis=-1, keepdims=True)


def _tc1_body(cnta_ref, cntb_ref, x_ref, w_ref, o_ref):
    dinv = lax.rsqrt(cnta_ref[...] + cntb_ref[...] + 1.0)
    o_ref[...] = jnp.dot(x_ref[...], w_ref[...],
                         preferred_element_type=jnp.float32) * dinv


def _tc2_body(part_ref, cnta_ref, cntb_ref, b1_ref, w2_ref, o_ref):
    dinv = lax.rsqrt(cnta_ref[...] + cntb_ref[...] + 1.0)
    t = (part_ref[0] + part_ref[1]) * dinv + b1_ref[...]
    t = jnp.maximum(t, 0.0)
    o_ref[...] = jnp.dot(t, w2_ref[...],
                         preferred_element_type=jnp.float32) * dinv


def _tc3_body(part_ref, cnta_ref, cntb_ref, b2_ref, o_ref):
    dinv = lax.rsqrt(cnta_ref[...] + cntb_ref[...] + 1.0)
    o_ref[...] = (part_ref[0] + part_ref[1]) * dinv + b2_ref[...]


def kernel(x, edge_index, edge_label_index, W1, b1, W2, b2):
    N, C = x.shape
    E = edge_index.shape[1]
    P = edge_label_index.shape[1]
    n_padr = (-(-N // N_S) + 127) // 128 * 128 * N_S
    B = 1024
    grid = (n_padr // B,)

    quant = ECH * N_BUF
    t_ps = -(-E // (N_W * quant)) * quant
    e_pad = N_W * t_ps - E
    pad_row = (-(-N // N_S) + 127) // 128 * 128 * N_S - 1
    src_f = jnp.concatenate(
        [edge_index[0], jnp.zeros((e_pad,), jnp.int32)])
    dst_f = jnp.concatenate(
        [edge_index[1], jnp.full((e_pad,), pad_row, jnp.int32)])
    E_p = N_W * t_ps
    dst3 = dst_f.reshape(N_W, t_ps // ECH, ECH)
    ea, eb = edge_label_index[0], edge_label_index[1]

    e0ps = int(round(2 * t_ps * 0.70 / quant)) * quant
    e1ps = 2 * t_ps - e0ps

    deg_call, n_pad = _make_deg(N, E_p)
    agg_call = _make_agg(N, e0ps, e1ps, C)
    dec_call = _make_decode(N, P, C)

    ones_e = jnp.ones((ECH,), jnp.float32)
    zseg = jnp.zeros((n_pad // N_S,), jnp.float32)

    cnt = deg_call(dst3, ones_e, zseg).reshape(N_C, n_pad)
    cnta = cnt[0].reshape(n_pad, 1)
    cntb = cnt[1].reshape(n_pad, 1)

    col = pl.BlockSpec((B, 1), lambda i: (i, 0))
    mat = pl.BlockSpec((B, C), lambda i: (i, 0))
    wts = pl.BlockSpec((C, C), lambda i: (0, 0))
    bias = pl.BlockSpec((1, C), lambda i: (0, 0))
    parts = pl.BlockSpec((N_C, B, C), lambda i: (0, i, 0))
    out_sds = jax.ShapeDtypeStruct((n_padr, C), jnp.float32)

    g1 = pl.pallas_call(
        _tc1_body, grid=grid,
        in_specs=[col, col, mat, wts],
        out_specs=mat, out_shape=out_sds,
    )(cnta, cntb, x, W1)

    part1 = agg_call(g1, src_f, dst_f)

    g2 = pl.pallas_call(
        _tc2_body, grid=grid,
        in_specs=[parts, col, col, bias, wts],
        out_specs=mat, out_shape=out_sds,
    )(part1, cnta, cntb, b1.reshape(1, C), W2)

    part2 = agg_call(g2, src_f, dst_f)

    z = pl.pallas_call(
        _tc3_body, grid=grid,
        in_specs=[parts, col, col, bias],
        out_specs=mat, out_shape=out_sds,
    )(part2, cnta, cntb, b2.reshape(1, C))

    partial_dots = dec_call(z, ea, eb)

    BP = 2000
    scores = pl.pallas_call(
        _tc4_body, grid=(P // BP,),
        in_specs=[pl.BlockSpec((BP, 16), lambda i: (i, 0))],
        out_specs=pl.BlockSpec((BP, 1), lambda i: (i, 0)),
        out_shape=jax.ShapeDtypeStruct((P, 1), jnp.float32),
    )(partial_dots)
    return scores.reshape(P)
